# baseline (speedup 1.0000x reference)
import functools

import jax
import jax.numpy as jnp
from jax import lax
from jax.experimental import pallas as pl
from jax.experimental.pallas import tpu as pltpu
from jax.experimental.pallas import tpu_sc as plsc

N = 10000
E = 320000
R = 8
D_IN = 128
D_T = 20
D_H = 128
D_CAT = R * D_H + D_H
NB = N * R
NB_PAD = 81920

NC = 2
NS = 16
NW = NC * NS
L = 16

B = 16
NCHUNK = E // B
NPT = NCHUNK // NS
PAIRS = (NPT + 1) // 2
IDXC = 32
E_PAD = E + IDXC * B
EPW = E // NW
CLOAD = 2000

ROW_BLK = 2000


def _mm_body(x_ref, nt_ref, emb_ref, wx_ref, wt_ref, b_ref, hall_ref, root_ref):
    t = jnp.dot(emb_ref[...], wt_ref[...], preferred_element_type=jnp.float32)
    h = jnp.dot(x_ref[...], wx_ref[...], preferred_element_type=jnp.float32)
    nt = nt_ref[...]
    for k in range(3):
        mask = (nt == k).astype(jnp.float32)
        h = h + mask * t[k][None, :]
    h = h + b_ref[...]
    hall_ref[...] = h[:, : R * D_H].astype(jnp.bfloat16)
    root_ref[...] = h[:, R * D_H :]


CROWS = NB_PAD // D_H


def _count_body(dtidx_hbm, out_hbm, tab, tbuf):
    cid = lax.axis_index("c")
    sid = lax.axis_index("s")
    w = sid * NC + cid
    one16 = jnp.ones((L,), jnp.float32)
    zero16 = jnp.zeros((L,), jnp.float32)

    def zbody(i, carry):
        tab[i // (D_H // L), pl.ds((i % (D_H // L)) * L, L)] = zero16
        return carry

    lax.fori_loop(0, CROWS * D_H // L, zbody, 0)

    base = w * EPW

    def lbody(q, carry):
        pltpu.sync_copy(dtidx_hbm.at[pl.ds(base + q * CLOAD, CLOAD)], tbuf)

        def jbody(j, carry2):
            dt = tbuf[pl.ds(j * L, L)]
            plsc.addupdate_scatter(
                tab, [lax.shift_right_logical(dt, 7), jnp.bitwise_and(dt, 127)],
                one16)
            return carry2

        lax.fori_loop(0, CLOAD // L, jbody, 0)
        return carry

    lax.fori_loop(0, EPW // CLOAD, lbody, 0)

    pltpu.sync_copy(tab, out_hbm.at[w])


def _scale_body(c_ref, s_ref):
    s_ref[...] = 1.0 / jnp.maximum(jnp.sum(c_ref[...], axis=0), 1.0)


NH = N // NC
ACC_ROWS = NH + 8
CC_FULL = NCHUNK // NS
CC_REM = NCHUNK - CC_FULL * NS


def _main_body(hflat_hbm, row_hbm, dtidx_hbm, scale_hbm, out_hbm,
               scale_tab, mbuf, fbuf, rbuf, tbuf, sbuf, dbuf, acc_ref,
               gsem, gsem2, ssem, ssem2):
    cid = lax.axis_index("c")
    sid = lax.axis_index("s")

    zero16 = jnp.zeros((L,), jnp.float32)

    def zm(i, carry):
        fbuf[0, i // (D_H // L), pl.ds((i % (D_H // L)) * L, L)] = zero16
        return carry

    lax.fori_loop(0, B * D_H // L, zm, 0)

    zbase = sid * (ACC_ROWS // NS)
    nz = ACC_ROWS // NS // L + 1

    def zacc(k, carry):
        off = jnp.minimum(k * L, ACC_ROWS // NS - L)
        pltpu.sync_copy(fbuf.at[0, pl.ds(0, L)], acc_ref.at[pl.ds(zbase + off, L)])
        return carry

    lax.fori_loop(0, nz, zacc, 0)

    iota = lax.iota(jnp.int32, L)

    def sload(c, carry):
        rbuf[pl.ds(0, L)] = iota + c * L
        pltpu.async_copy(scale_hbm.at[rbuf.at[pl.ds(0, L)]],
                         scale_tab.at[pl.ds(c * L, L)], gsem).wait()
        return carry

    lax.fori_loop(0, CROWS // L, sload, 0)
    plsc.subcore_barrier()

    lo = cid * NH
    ebase0 = sid * NPT * B

    def scale_and_scatter(lc, buf, valid, sem):
        for j in range(B // L):
            dtv = tbuf[pl.ds(lc * B + j * L, L)]
            sbuf[pl.ds(j * L, L)] = plsc.load_gather(
                scale_tab,
                [lax.shift_right_logical(dtv, 7), jnp.bitwise_and(dtv, 127)])
            dstv = lax.shift_right_logical(dtv, 3).astype(jnp.int32) - lo
            owned = (dstv >= 0) & (dstv < NH)
            if valid is not None:
                owned = owned & valid
            dbuf[buf, pl.ds(j * L, L)] = jnp.where(owned, dstv, NH)

        def gbody(g, carry):
            sv = sbuf[pl.ds(g * L, L)]
            for i in range(L):
                s = sv[i]
                rr = g * L + i
                for k in range(D_H // (2 * L)):
                    pk = plsc.bitcast(mbuf[buf, rr, pl.ds(k * L, L)],
                                      jnp.bfloat16)
                    a, b = plsc.unpack(pk, format=plsc.PackFormat.INTERLEAVED)
                    fbuf[buf, rr, pl.ds(k * 2 * L, L)] = a * s
                    fbuf[buf, rr, pl.ds(k * 2 * L + L, L)] = b * s
            return carry

        lax.fori_loop(0, B // L, gbody, 0)
        pltpu.async_copy(fbuf.at[buf], acc_ref.at[dbuf.at[buf]], sem, add=True)

    def drain_scatters():
        pltpu.make_async_copy(fbuf.at[0], acc_ref.at[dbuf.at[0]], ssem).wait()
        pltpu.make_async_copy(fbuf.at[1], acc_ref.at[dbuf.at[1]], ssem2).wait()

    def pair(p, carry):
        @pl.when(p % (IDXC // 2) == 0)
        def _():
            off = ebase0 + (p // (IDXC // 2)) * (IDXC * B)
            pltpu.sync_copy(row_hbm.at[pl.ds(off, IDXC * B)], rbuf)
            pltpu.sync_copy(dtidx_hbm.at[pl.ds(off, IDXC * B)], tbuf)

        @pl.when(p > 0)
        def _():
            drain_scatters()

        lc0 = (p % (IDXC // 2)) * 2
        lc1 = lc0 + 1
        ca = pltpu.async_copy(hflat_hbm.at[rbuf.at[pl.ds(lc0 * B, B)]],
                              mbuf.at[0], gsem)
        cb = pltpu.async_copy(hflat_hbm.at[rbuf.at[pl.ds(lc1 * B, B)]],
                              mbuf.at[1], gsem2)
        ca.wait()
        scale_and_scatter(lc0, 0, None, ssem)
        cb.wait()
        scale_and_scatter(lc1, 1, 2 * p + 1 < NPT, ssem2)
        return carry

    lax.fori_loop(0, PAIRS, pair, 0)
    drain_scatters()

    plsc.subcore_barrier()
    fbase = sid * 312
    frows = jnp.where(sid == NS - 1, 320, 312)

    def facc(k, carry):
        off = jnp.minimum(k * L, frows - L)
        pltpu.sync_copy(acc_ref.at[pl.ds(fbase + off, L)],
                        out_hbm.at[pl.ds(lo + fbase + off, L)])
        return carry

    lax.fori_loop(0, 320 // L, facc, 0)


def _ep_body(p_ref, root_ref, o_ref):
    o_ref[...] = jnp.maximum(p_ref[...] + root_ref[...], 0.0)


def kernel(x, node_type, edge_index, edge_type, node_emb, W, W_root, bias):
    src = edge_index[0].astype(jnp.int32)
    dst = edge_index[1].astype(jnp.int32)
    et = edge_type.astype(jnp.int32)
    rowidx = src * R + et
    dtidx = dst * R + et
    zpad = jnp.zeros((E_PAD - E,), jnp.int32)
    rowidx_p = jnp.concatenate([rowidx, zpad])
    dtidx_p = jnp.concatenate([dtidx, zpad])

    cperm = (jnp.arange(R * D_H) // 32) * 32 + jnp.tile(
        jnp.arange(32).reshape(2, 16).T.reshape(32), R * D_H // 32)
    wx_rel = jnp.transpose(W[:, :D_IN, :], (1, 0, 2)).reshape(D_IN, R * D_H)
    wt_rel = jnp.transpose(W[:, D_IN:, :], (1, 0, 2)).reshape(D_T, R * D_H)
    wx = jnp.concatenate([wx_rel[:, cperm], W_root[:D_IN]], axis=1)
    wt = jnp.concatenate([wt_rel[:, cperm], W_root[D_IN:]], axis=1)
    bias_full = jnp.concatenate(
        [jnp.zeros((R * D_H,), jnp.float32), bias]).reshape(1, D_CAT)
    nt2 = node_type.astype(jnp.int32).reshape(N, 1)

    hall, root = pl.pallas_call(
        _mm_body,
        grid=(N // ROW_BLK,),
        in_specs=[
            pl.BlockSpec((ROW_BLK, D_IN), lambda i: (i, 0)),
            pl.BlockSpec((ROW_BLK, 1), lambda i: (i, 0)),
            pl.BlockSpec((3, D_T), lambda i: (0, 0)),
            pl.BlockSpec((D_IN, D_CAT), lambda i: (0, 0)),
            pl.BlockSpec((D_T, D_CAT), lambda i: (0, 0)),
            pl.BlockSpec((1, D_CAT), lambda i: (0, 0)),
        ],
        out_specs=[
            pl.BlockSpec((ROW_BLK, R * D_H), lambda i: (i, 0)),
            pl.BlockSpec((ROW_BLK, D_H), lambda i: (i, 0)),
        ],
        out_shape=[
            jax.ShapeDtypeStruct((N, R * D_H), jnp.bfloat16),
            jax.ShapeDtypeStruct((N, D_H), jnp.float32),
        ],
    )(x, nt2, node_emb, wx, wt, bias_full)

    hflat = jax.lax.bitcast_convert_type(
        hall.reshape(N * R, D_H // 2, 2), jnp.int32)

    mesh = plsc.VectorSubcoreMesh(
        core_axis_name="c", subcore_axis_name="s",
        num_cores=NC, num_subcores=NS)

    sc_params = pltpu.CompilerParams(needs_layout_passes=False)
    count_k = pl.kernel(
        _count_body,
        out_type=jax.ShapeDtypeStruct((NW, CROWS, D_H), jnp.float32),
        mesh=mesh,
        compiler_params=sc_params,
        scratch_types=[
            pltpu.VMEM((CROWS, D_H), jnp.float32),
            pltpu.VMEM((CLOAD,), jnp.int32),
        ],
    )
    cnt = count_k(dtidx)

    scale = pl.pallas_call(
        _scale_body,
        grid=(CROWS // 64,),
        in_specs=[pl.BlockSpec((NW, 64, D_H), lambda i: (0, i, 0))],
        out_specs=pl.BlockSpec((64, D_H), lambda i: (i, 0)),
        out_shape=jax.ShapeDtypeStruct((CROWS, D_H), jnp.float32),
    )(cnt)

    main_k = pl.kernel(
        _main_body,
        out_type=jax.ShapeDtypeStruct((N, D_H), jnp.float32),
        mesh=mesh,
        compiler_params=pltpu.CompilerParams(
            needs_layout_passes=False, use_tc_tiling_on_sc=False),
        scratch_types=[
            pltpu.VMEM((CROWS, D_H), jnp.float32),
            pltpu.VMEM((2, B, D_H // 2), jnp.int32),
            pltpu.VMEM((2, B, D_H), jnp.float32),
            pltpu.VMEM((IDXC * B,), jnp.int32),
            pltpu.VMEM((IDXC * B,), jnp.int32),
            pltpu.VMEM((B,), jnp.float32),
            pltpu.VMEM((2, B), jnp.int32),
            pltpu.VMEM_SHARED((ACC_ROWS, D_H), jnp.float32),
            pltpu.SemaphoreType.DMA,
            pltpu.SemaphoreType.DMA,
            pltpu.SemaphoreType.DMA,
            pltpu.SemaphoreType.DMA,
        ],
    )
    part = main_k(hflat, rowidx_p, dtidx_p, scale)

    out = pl.pallas_call(
        _ep_body,
        grid=(N // ROW_BLK,),
        in_specs=[
            pl.BlockSpec((ROW_BLK, D_H), lambda i: (i, 0)),
            pl.BlockSpec((ROW_BLK, D_H), lambda i: (i, 0)),
        ],
        out_specs=pl.BlockSpec((ROW_BLK, D_H), lambda i: (i, 0)),
        out_shape=jax.ShapeDtypeStruct((N, D_H), jnp.float32),
    )(part, root)
    return (out,)


# speedup vs baseline: 9.4929x; 9.4929x over previous
import functools

import jax
import jax.numpy as jnp
from jax import lax
from jax.experimental import pallas as pl
from jax.experimental.pallas import tpu as pltpu
from jax.experimental.pallas import tpu_sc as plsc

N = 10000
E = 320000
R = 8
D_IN = 128
D_T = 20
D_H = 128
D_CAT = R * D_H + D_H
NB = N * R
NB_PAD = 81920

NC = 2
NS = 16
NW = NC * NS
L = 16

B = 32
IDXC = 4
EPW = E // NW
CLOAD = 2000
CAP = 10240

ROW_BLK = 2000


def _mm_body(x_ref, nt_ref, emb_ref, wx_ref, wt_ref, b_ref, hall_ref, root_ref):
    t = jnp.dot(emb_ref[...], wt_ref[...], preferred_element_type=jnp.float32)
    h = jnp.dot(x_ref[...], wx_ref[...], preferred_element_type=jnp.float32)
    nt = nt_ref[...]
    for k in range(3):
        mask = (nt == k).astype(jnp.float32)
        h = h + mask * t[k][None, :]
    h = h + b_ref[...]
    hall_ref[...] = h[:, : R * D_H]
    root_ref[...] = h[:, R * D_H :]


CROWS = NB_PAD // D_H


def _count_body(dtidx_hbm, out_hbm, tab, tbuf):
    cid = lax.axis_index("c")
    sid = lax.axis_index("s")
    w = sid * NC + cid
    one16 = jnp.ones((L,), jnp.float32)
    zero16 = jnp.zeros((L,), jnp.float32)

    def zbody(i, carry):
        tab[i // (D_H // L), pl.ds((i % (D_H // L)) * L, L)] = zero16
        return carry

    lax.fori_loop(0, CROWS * D_H // L, zbody, 0)

    base = w * EPW

    def lbody(q, carry):
        pltpu.sync_copy(dtidx_hbm.at[pl.ds(base + q * CLOAD, CLOAD)], tbuf)

        def jbody(j, carry2):
            dt = tbuf[pl.ds(j * L, L)]
            plsc.addupdate_scatter(
                tab, [lax.shift_right_logical(dt, 7), jnp.bitwise_and(dt, 127)],
                one16)
            return carry2

        lax.fori_loop(0, CLOAD // L, jbody, 0)
        return carry

    lax.fori_loop(0, EPW // CLOAD, lbody, 0)

    pltpu.sync_copy(tab, out_hbm.at[w])


def _scale_body(c_ref, s_ref):
    s_ref[...] = 1.0 / jnp.maximum(jnp.sum(c_ref[...], axis=0), 1.0)


def _part_body(row_hbm, dtidx_hbm, rout_hbm, tout_hbm, cnt_hbm,
               rin, tin, r0, t0, r1, t1, cb):
    cid = lax.axis_index("c")
    sid = lax.axis_index("s")
    w = sid * NC + cid
    base = w * EPW
    zero16i = jnp.zeros((L,), jnp.int32)

    def zb(i, carry):
        r0[pl.ds(i * L, L)] = zero16i
        t0[pl.ds(i * L, L)] = zero16i
        r1[pl.ds(i * L, L)] = zero16i
        t1[pl.ds(i * L, L)] = zero16i
        return carry

    lax.fori_loop(0, CAP // L, zb, 0)

    def load_q(q, offs):
        pltpu.sync_copy(row_hbm.at[pl.ds(base + q * CLOAD, CLOAD)], rin)
        pltpu.sync_copy(dtidx_hbm.at[pl.ds(base + q * CLOAD, CLOAD)], tin)

        def jb(j, offs2):
            off0, off1 = offs2
            rv = rin[pl.ds(j * L, L)]
            tv = tin[pl.ds(j * L, L)]
            m0 = lax.shift_right_logical(tv, 3) < NH
            cnt0 = plsc.all_reduce_population_count(m0)[0]
            plsc.store_compressed(r0.at[pl.ds(off0, L)], rv, mask=m0)
            plsc.store_compressed(t0.at[pl.ds(off0, L)], tv, mask=m0)
            m1 = jnp.logical_not(m0)
            plsc.store_compressed(r1.at[pl.ds(off1, L)], rv, mask=m1)
            plsc.store_compressed(t1.at[pl.ds(off1, L)], tv, mask=m1)
            return (off0 + cnt0, off1 + (L - cnt0))

        return lax.fori_loop(0, CLOAD // L, jb, offs)

    n0, n1 = lax.fori_loop(0, EPW // CLOAD, load_q,
                           (jnp.int32(0), jnp.int32(0)))
    iota = lax.iota(jnp.int32, L)
    for j in range(D_H // L):
        cb[pl.ds(j * L, L)] = jnp.zeros((L,), jnp.int32)
    cb[pl.ds(0, L)] = jnp.where(iota == 0, n0, jnp.where(iota == 1, n1, 0))
    pltpu.sync_copy(cb, cnt_hbm.at[w * 8])
    pltpu.sync_copy(r0, rout_hbm.at[pl.ds(w * CAP, CAP)])
    pltpu.sync_copy(t0, tout_hbm.at[pl.ds(w * CAP, CAP)])
    pltpu.sync_copy(r1, rout_hbm.at[pl.ds((NW + w) * CAP, CAP)])
    pltpu.sync_copy(t1, tout_hbm.at[pl.ds((NW + w) * CAP, CAP)])


NH = N // NC
ACC_ROWS = NH + 8


def _main_body(hflat_hbm, rpart_hbm, tpart_hbm, cnt_hbm, scale_hbm, out_hbm,
               scale_tab, mbuf, rbuf, tbuf, sbuf, dbuf, cb, acc_ref,
               gsem, gsem2, ssem, ssem2):
    cid = lax.axis_index("c")
    sid = lax.axis_index("s")

    zero16 = jnp.zeros((L,), jnp.float32)

    def zm(i, carry):
        mbuf[0, i // (D_H // L), pl.ds((i % (D_H // L)) * L, L)] = zero16
        return carry

    lax.fori_loop(0, B * D_H // L, zm, 0)

    zbase = sid * (ACC_ROWS // NS)
    nz = ACC_ROWS // NS // L + 1

    def zacc(k, carry):
        off = jnp.minimum(k * L, ACC_ROWS // NS - L)
        pltpu.sync_copy(mbuf.at[0, pl.ds(0, L)], acc_ref.at[pl.ds(zbase + off, L)])
        return carry

    lax.fori_loop(0, nz, zacc, 0)

    iota = lax.iota(jnp.int32, L)

    def sload(c, carry):
        rbuf[pl.ds(0, L)] = iota + c * L
        pltpu.async_copy(scale_hbm.at[rbuf.at[pl.ds(0, L)]],
                         scale_tab.at[pl.ds(c * L, L)], gsem).wait()
        return carry

    lax.fori_loop(0, CROWS // L, sload, 0)
    plsc.subcore_barrier()

    lo = cid * NH
    iota = lax.iota(jnp.int32, L)
    rbuf[pl.ds(0, L)] = iota * 8 + 16 * sid
    pltpu.async_copy(cnt_hbm.at[rbuf.at[pl.ds(0, 2)]], cb, gsem).wait()
    cv0 = cb[0, pl.ds(0, L)]
    cv1 = cb[1, pl.ds(0, L)]

    def scale_and_scatter(lc, buf, ebase, n, sem):
        for j in range(B // L):
            dtv = tbuf[pl.ds(lc * B + j * L, L)]
            sbuf[pl.ds(j * L, L)] = plsc.load_gather(
                scale_tab,
                [lax.shift_right_logical(dtv, 7), jnp.bitwise_and(dtv, 127)])
            dstv = lax.shift_right_logical(dtv, 3).astype(jnp.int32) - lo
            owned = (dstv >= 0) & (dstv < NH)
            owned = owned & ((ebase + j * L + iota) < n)
            dbuf[buf, pl.ds(j * L, L)] = jnp.where(owned, dstv, NH)

        def gbody(g, carry):
            sv = sbuf[pl.ds(g * L, L)]
            for i in range(L):
                s = sv[i]
                rr = g * L + i
                for k in range(D_H // L):
                    mbuf[buf, rr, pl.ds(k * L, L)] = (
                        mbuf[buf, rr, pl.ds(k * L, L)] * s)
            return carry

        lax.fori_loop(0, B // L, gbody, 0)
        pltpu.async_copy(mbuf.at[buf], acc_ref.at[dbuf.at[buf]], sem, add=True)

    def drain_scatters():
        pltpu.make_async_copy(mbuf.at[0], acc_ref.at[dbuf.at[0]], ssem).wait()
        pltpu.make_async_copy(mbuf.at[1], acc_ref.at[dbuf.at[1]], ssem2).wait()

    def region(r01, carry):
        reg = 2 * sid + r01
        n0 = jnp.where(cid == 0, cv0[0], cv0[1])
        n1 = jnp.where(cid == 0, cv1[0], cv1[1])
        n = jnp.where(r01 == 0, n0, n1)
        npairs = (n + 2 * B - 1) // (2 * B)

        def pair(p, carry2):
            @pl.when(p % (IDXC // 2) == 0)
            def _():
                off = ((cid * NW + reg) * CAP
                       + (p // (IDXC // 2)) * (IDXC * B))
                pltpu.sync_copy(rpart_hbm.at[pl.ds(off, IDXC * B)], rbuf)
                pltpu.sync_copy(tpart_hbm.at[pl.ds(off, IDXC * B)], tbuf)

            @pl.when(carry2 + p > 0)
            def _():
                drain_scatters()

            lc0 = (p % (IDXC // 2)) * 2
            lc1 = lc0 + 1
            ca = pltpu.async_copy(hflat_hbm.at[rbuf.at[pl.ds(lc0 * B, B)]],
                                  mbuf.at[0], gsem)
            cb2 = pltpu.async_copy(hflat_hbm.at[rbuf.at[pl.ds(lc1 * B, B)]],
                                   mbuf.at[1], gsem2)
            ca.wait()
            scale_and_scatter(lc0, 0, 2 * p * B, n, ssem)
            cb2.wait()
            scale_and_scatter(lc1, 1, (2 * p + 1) * B, n, ssem2)
            return carry2 + 1

        return lax.fori_loop(0, npairs, pair, carry)

    done = lax.fori_loop(0, 2, region, jnp.int32(0))

    @pl.when(done > 0)
    def _():
        drain_scatters()


    plsc.subcore_barrier()
    fbase = sid * 312
    frows = jnp.where(sid == NS - 1, 320, 312)

    def facc(k, carry):
        off = jnp.minimum(k * 8, frows - 8)
        pltpu.sync_copy(acc_ref.at[pl.ds(fbase + off, 8)],
                        out_hbm.at[pl.ds(lo + fbase + off, 8)])
        return carry

    lax.fori_loop(0, 320 // 8, facc, 0)


def _ep_body(p_ref, root_ref, o_ref):
    o_ref[...] = jnp.maximum(p_ref[...] + root_ref[...], 0.0)


def kernel(x, node_type, edge_index, edge_type, node_emb, W, W_root, bias):
    src = edge_index[0].astype(jnp.int32)
    dst = edge_index[1].astype(jnp.int32)
    et = edge_type.astype(jnp.int32)
    rowidx = src * R + et
    dtidx = dst * R + et

    wx = jnp.concatenate(
        [jnp.transpose(W[:, :D_IN, :], (1, 0, 2)).reshape(D_IN, R * D_H),
         W_root[:D_IN]], axis=1)
    wt = jnp.concatenate(
        [jnp.transpose(W[:, D_IN:, :], (1, 0, 2)).reshape(D_T, R * D_H),
         W_root[D_IN:]], axis=1)
    bias_full = jnp.concatenate(
        [jnp.zeros((R * D_H,), jnp.float32), bias]).reshape(1, D_CAT)
    nt2 = node_type.astype(jnp.int32).reshape(N, 1)

    hall, root = pl.pallas_call(
        _mm_body,
        grid=(N // ROW_BLK,),
        in_specs=[
            pl.BlockSpec((ROW_BLK, D_IN), lambda i: (i, 0)),
            pl.BlockSpec((ROW_BLK, 1), lambda i: (i, 0)),
            pl.BlockSpec((3, D_T), lambda i: (0, 0)),
            pl.BlockSpec((D_IN, D_CAT), lambda i: (0, 0)),
            pl.BlockSpec((D_T, D_CAT), lambda i: (0, 0)),
            pl.BlockSpec((1, D_CAT), lambda i: (0, 0)),
        ],
        out_specs=[
            pl.BlockSpec((ROW_BLK, R * D_H), lambda i: (i, 0)),
            pl.BlockSpec((ROW_BLK, D_H), lambda i: (i, 0)),
        ],
        out_shape=[
            jax.ShapeDtypeStruct((N, R * D_H), jnp.float32),
            jax.ShapeDtypeStruct((N, D_H), jnp.float32),
        ],
    )(x, nt2, node_emb, wx, wt, bias_full)

    hflat = hall.reshape(N * R, D_H)

    mesh = plsc.VectorSubcoreMesh(
        core_axis_name="c", subcore_axis_name="s",
        num_cores=NC, num_subcores=NS)

    sc_params = pltpu.CompilerParams(needs_layout_passes=False)
    count_k = pl.kernel(
        _count_body,
        out_type=jax.ShapeDtypeStruct((NW, CROWS, D_H), jnp.float32),
        mesh=mesh,
        compiler_params=sc_params,
        scratch_types=[
            pltpu.VMEM((CROWS, D_H), jnp.float32),
            pltpu.VMEM((CLOAD,), jnp.int32),
        ],
    )
    cnt = count_k(dtidx)

    part_k = pl.kernel(
        _part_body,
        out_type=[
            jax.ShapeDtypeStruct((NC * NW * CAP,), jnp.int32),
            jax.ShapeDtypeStruct((NC * NW * CAP,), jnp.int32),
            jax.ShapeDtypeStruct((NW * 8, D_H), jnp.int32),
        ],
        mesh=mesh,
        compiler_params=sc_params,
        scratch_types=[
            pltpu.VMEM((CLOAD,), jnp.int32),
            pltpu.VMEM((CLOAD,), jnp.int32),
            pltpu.VMEM((CAP,), jnp.int32),
            pltpu.VMEM((CAP,), jnp.int32),
            pltpu.VMEM((CAP,), jnp.int32),
            pltpu.VMEM((CAP,), jnp.int32),
            pltpu.VMEM((D_H,), jnp.int32),
        ],
    )
    rpart, tpart, ecnt = part_k(rowidx, dtidx)

    scale = pl.pallas_call(
        _scale_body,
        grid=(CROWS // 64,),
        in_specs=[pl.BlockSpec((NW, 64, D_H), lambda i: (0, i, 0))],
        out_specs=pl.BlockSpec((64, D_H), lambda i: (i, 0)),
        out_shape=jax.ShapeDtypeStruct((CROWS, D_H), jnp.float32),
    )(cnt)

    main_k = pl.kernel(
        _main_body,
        out_type=jax.ShapeDtypeStruct((N, D_H), jnp.float32),
        mesh=mesh,
        compiler_params=sc_params,
        scratch_types=[
            pltpu.VMEM((CROWS, D_H), jnp.float32),
            pltpu.VMEM((2, B, D_H), jnp.float32),
            pltpu.VMEM((IDXC * B,), jnp.int32),
            pltpu.VMEM((IDXC * B,), jnp.int32),
            pltpu.VMEM((B,), jnp.float32),
            pltpu.VMEM((2, B), jnp.int32),
            pltpu.VMEM((2, D_H), jnp.int32),
            pltpu.VMEM_SHARED((ACC_ROWS, D_H), jnp.float32),
            pltpu.SemaphoreType.DMA,
            pltpu.SemaphoreType.DMA,
            pltpu.SemaphoreType.DMA,
            pltpu.SemaphoreType.DMA,
        ],
    )
    part = main_k(hflat, rpart, tpart, ecnt, scale)

    out = pl.pallas_call(
        _ep_body,
        grid=(N // ROW_BLK,),
        in_specs=[
            pl.BlockSpec((ROW_BLK, D_H), lambda i: (i, 0)),
            pl.BlockSpec((ROW_BLK, D_H), lambda i: (i, 0)),
        ],
        out_specs=pl.BlockSpec((ROW_BLK, D_H), lambda i: (i, 0)),
        out_shape=jax.ShapeDtypeStruct((N, D_H), jnp.float32),
    )(part, root)
    return (out,)


# speedup vs baseline: 9.5545x; 1.0065x over previous
import functools

import jax
import jax.numpy as jnp
from jax import lax
from jax.experimental import pallas as pl
from jax.experimental.pallas import tpu as pltpu
from jax.experimental.pallas import tpu_sc as plsc

N = 10000
E = 320000
R = 8
D_IN = 128
D_T = 20
D_H = 128
D_CAT = R * D_H + D_H
NB = N * R
NB_PAD = 81920

NC = 2
NS = 16
NW = NC * NS
L = 16

B = 32
IDXC = 4
EPW = E // NW
CLOAD = 2000
CAP = 10240

ROW_BLK = 2000


def _mm_body(x_ref, nt_ref, emb_ref, wx_ref, wt_ref, b_ref, hall_ref, root_ref):
    t = jnp.dot(emb_ref[...], wt_ref[...], preferred_element_type=jnp.float32)
    h = jnp.dot(x_ref[...], wx_ref[...], preferred_element_type=jnp.float32)
    nt = nt_ref[...]
    for k in range(3):
        mask = (nt == k).astype(jnp.float32)
        h = h + mask * t[k][None, :]
    h = h + b_ref[...]
    hall_ref[...] = h[:, : R * D_H]
    root_ref[...] = h[:, R * D_H :]


CROWS = NB_PAD // D_H


def _count_body(dtidx_hbm, out_hbm, tab, tbuf):
    cid = lax.axis_index("c")
    sid = lax.axis_index("s")
    w = sid * NC + cid
    one16 = jnp.ones((L,), jnp.float32)
    zero16 = jnp.zeros((L,), jnp.float32)

    def zbody(i, carry):
        tab[i // (D_H // L), pl.ds((i % (D_H // L)) * L, L)] = zero16
        return carry

    lax.fori_loop(0, CROWS * D_H // L, zbody, 0)

    base = w * EPW

    def lbody(q, carry):
        pltpu.sync_copy(dtidx_hbm.at[pl.ds(base + q * CLOAD, CLOAD)], tbuf)

        def jbody(j, carry2):
            dt = tbuf[pl.ds(j * L, L)]
            plsc.addupdate_scatter(
                tab, [lax.shift_right_logical(dt, 7), jnp.bitwise_and(dt, 127)],
                one16)
            return carry2

        lax.fori_loop(0, CLOAD // L, jbody, 0)
        return carry

    lax.fori_loop(0, EPW // CLOAD, lbody, 0)

    pltpu.sync_copy(tab, out_hbm.at[w])


def _scale_body(c_ref, s_ref):
    s_ref[...] = 1.0 / jnp.maximum(jnp.sum(c_ref[...], axis=0), 1.0)


def _part_body(row_hbm, dtidx_hbm, rout_hbm, tout_hbm, cnt_hbm, cntp_hbm,
               rin, tin, r0, t0, r1, t1, cb, tab):
    cid = lax.axis_index("c")
    sid = lax.axis_index("s")
    w = sid * NC + cid
    base = w * EPW
    zero16i = jnp.zeros((L,), jnp.int32)
    zero16 = jnp.zeros((L,), jnp.float32)
    one16 = jnp.ones((L,), jnp.float32)

    def zb(i, carry):
        r0[pl.ds(i * L, L)] = zero16i
        t0[pl.ds(i * L, L)] = zero16i
        r1[pl.ds(i * L, L)] = zero16i
        t1[pl.ds(i * L, L)] = zero16i
        return carry

    lax.fori_loop(0, CAP // L, zb, 0)

    def zt(i, carry):
        tab[i // (D_H // L), pl.ds((i % (D_H // L)) * L, L)] = zero16
        return carry

    lax.fori_loop(0, CROWS * D_H // L, zt, 0)

    def load_q(q, offs):
        pltpu.sync_copy(row_hbm.at[pl.ds(base + q * CLOAD, CLOAD)], rin)
        pltpu.sync_copy(dtidx_hbm.at[pl.ds(base + q * CLOAD, CLOAD)], tin)

        def jb(j, offs2):
            off0, off1 = offs2
            rv = rin[pl.ds(j * L, L)]
            tv = tin[pl.ds(j * L, L)]
            plsc.addupdate_scatter(
                tab, [lax.shift_right_logical(tv, 7), jnp.bitwise_and(tv, 127)],
                one16)
            m0 = lax.shift_right_logical(tv, 3) < NH
            cnt0 = plsc.all_reduce_population_count(m0)[0]
            plsc.store_compressed(r0.at[pl.ds(off0, L)], rv, mask=m0)
            plsc.store_compressed(t0.at[pl.ds(off0, L)], tv, mask=m0)
            m1 = jnp.logical_not(m0)
            plsc.store_compressed(r1.at[pl.ds(off1, L)], rv, mask=m1)
            plsc.store_compressed(t1.at[pl.ds(off1, L)], tv, mask=m1)
            return (off0 + cnt0, off1 + (L - cnt0))

        return lax.fori_loop(0, CLOAD // L, jb, offs)

    n0, n1 = lax.fori_loop(0, EPW // CLOAD, load_q,
                           (jnp.int32(0), jnp.int32(0)))
    iota = lax.iota(jnp.int32, L)
    for j in range(D_H // L):
        cb[pl.ds(j * L, L)] = jnp.zeros((L,), jnp.int32)
    cb[pl.ds(0, L)] = jnp.where(iota == 0, n0, jnp.where(iota == 1, n1, 0))
    pltpu.sync_copy(cb, cnt_hbm.at[w * 8])
    pltpu.sync_copy(r0, rout_hbm.at[pl.ds(w * CAP, CAP)])
    pltpu.sync_copy(t0, tout_hbm.at[pl.ds(w * CAP, CAP)])
    pltpu.sync_copy(r1, rout_hbm.at[pl.ds((NW + w) * CAP, CAP)])
    pltpu.sync_copy(t1, tout_hbm.at[pl.ds((NW + w) * CAP, CAP)])
    pltpu.sync_copy(tab, cntp_hbm.at[w])


NH = N // NC
ACC_ROWS = NH + 8


def _main_body(hflat_hbm, rpart_hbm, tpart_hbm, cnt_hbm, scale_hbm, out_hbm,
               scale_tab, mbuf, rbuf, tbuf, sbuf, dbuf, cb, acc_ref,
               gsem, gsem2, ssem, ssem2):
    cid = lax.axis_index("c")
    sid = lax.axis_index("s")

    zero16 = jnp.zeros((L,), jnp.float32)

    def zm(i, carry):
        mbuf[0, i // (D_H // L), pl.ds((i % (D_H // L)) * L, L)] = zero16
        return carry

    lax.fori_loop(0, B * D_H // L, zm, 0)

    zbase = sid * (ACC_ROWS // NS)
    nz = ACC_ROWS // NS // L + 1

    def zacc(k, carry):
        off = jnp.minimum(k * L, ACC_ROWS // NS - L)
        pltpu.sync_copy(mbuf.at[0, pl.ds(0, L)], acc_ref.at[pl.ds(zbase + off, L)])
        return carry

    lax.fori_loop(0, nz, zacc, 0)

    iota = lax.iota(jnp.int32, L)

    def sload(c, carry):
        rbuf[pl.ds(0, L)] = iota + c * L
        pltpu.async_copy(scale_hbm.at[rbuf.at[pl.ds(0, L)]],
                         scale_tab.at[pl.ds(c * L, L)], gsem).wait()
        return carry

    lax.fori_loop(0, CROWS // L, sload, 0)
    plsc.subcore_barrier()

    lo = cid * NH
    iota = lax.iota(jnp.int32, L)
    rbuf[pl.ds(0, L)] = iota * 8 + 16 * sid
    pltpu.async_copy(cnt_hbm.at[rbuf.at[pl.ds(0, 2)]], cb, gsem).wait()
    cv0 = cb[0, pl.ds(0, L)]
    cv1 = cb[1, pl.ds(0, L)]

    def scale_and_scatter(lc, buf, ebase, n, sem):
        for j in range(B // L):
            dtv = tbuf[pl.ds(lc * B + j * L, L)]
            sbuf[pl.ds(j * L, L)] = plsc.load_gather(
                scale_tab,
                [lax.shift_right_logical(dtv, 7), jnp.bitwise_and(dtv, 127)])
            dstv = lax.shift_right_logical(dtv, 3).astype(jnp.int32) - lo
            owned = (dstv >= 0) & (dstv < NH)
            owned = owned & ((ebase + j * L + iota) < n)
            dbuf[buf, pl.ds(j * L, L)] = jnp.where(owned, dstv, NH)

        def gbody(g, carry):
            sv = sbuf[pl.ds(g * L, L)]
            for i in range(L):
                s = sv[i]
                rr = g * L + i
                for k in range(D_H // L):
                    mbuf[buf, rr, pl.ds(k * L, L)] = (
                        mbuf[buf, rr, pl.ds(k * L, L)] * s)
            return carry

        lax.fori_loop(0, B // L, gbody, 0)
        pltpu.async_copy(mbuf.at[buf], acc_ref.at[dbuf.at[buf]], sem, add=True)

    def drain_scatters():
        pltpu.make_async_copy(mbuf.at[0], acc_ref.at[dbuf.at[0]], ssem).wait()
        pltpu.make_async_copy(mbuf.at[1], acc_ref.at[dbuf.at[1]], ssem2).wait()

    def region(r01, carry):
        reg = 2 * sid + r01
        n0 = jnp.where(cid == 0, cv0[0], cv0[1])
        n1 = jnp.where(cid == 0, cv1[0], cv1[1])
        n = jnp.where(r01 == 0, n0, n1)
        npairs = (n + 2 * B - 1) // (2 * B)

        def pair(p, carry2):
            @pl.when(p % (IDXC // 2) == 0)
            def _():
                off = ((cid * NW + reg) * CAP
                       + (p // (IDXC // 2)) * (IDXC * B))
                pltpu.sync_copy(rpart_hbm.at[pl.ds(off, IDXC * B)], rbuf)
                pltpu.sync_copy(tpart_hbm.at[pl.ds(off, IDXC * B)], tbuf)

            @pl.when(carry2 + p > 0)
            def _():
                drain_scatters()

            lc0 = (p % (IDXC // 2)) * 2
            lc1 = lc0 + 1
            ca = pltpu.async_copy(hflat_hbm.at[rbuf.at[pl.ds(lc0 * B, B)]],
                                  mbuf.at[0], gsem)
            cb2 = pltpu.async_copy(hflat_hbm.at[rbuf.at[pl.ds(lc1 * B, B)]],
                                   mbuf.at[1], gsem2)
            ca.wait()
            scale_and_scatter(lc0, 0, 2 * p * B, n, ssem)
            cb2.wait()
            scale_and_scatter(lc1, 1, (2 * p + 1) * B, n, ssem2)
            return carry2 + 1

        return lax.fori_loop(0, npairs, pair, carry)

    done = lax.fori_loop(0, 2, region, jnp.int32(0))

    @pl.when(done > 0)
    def _():
        drain_scatters()


    plsc.subcore_barrier()
    fbase = sid * 312
    frows = jnp.where(sid == NS - 1, 320, 312)

    def facc(k, carry):
        off = jnp.minimum(k * 8, frows - 8)
        pltpu.sync_copy(acc_ref.at[pl.ds(fbase + off, 8)],
                        out_hbm.at[pl.ds(lo + fbase + off, 8)])
        return carry

    lax.fori_loop(0, 320 // 8, facc, 0)


def _ep_body(p_ref, root_ref, o_ref):
    o_ref[...] = jnp.maximum(p_ref[...] + root_ref[...], 0.0)


def kernel(x, node_type, edge_index, edge_type, node_emb, W, W_root, bias):
    src = edge_index[0].astype(jnp.int32)
    dst = edge_index[1].astype(jnp.int32)
    et = edge_type.astype(jnp.int32)
    rowidx = src * R + et
    dtidx = dst * R + et

    wx = jnp.concatenate(
        [jnp.transpose(W[:, :D_IN, :], (1, 0, 2)).reshape(D_IN, R * D_H),
         W_root[:D_IN]], axis=1)
    wt = jnp.concatenate(
        [jnp.transpose(W[:, D_IN:, :], (1, 0, 2)).reshape(D_T, R * D_H),
         W_root[D_IN:]], axis=1)
    bias_full = jnp.concatenate(
        [jnp.zeros((R * D_H,), jnp.float32), bias]).reshape(1, D_CAT)
    nt2 = node_type.astype(jnp.int32).reshape(N, 1)

    hall, root = pl.pallas_call(
        _mm_body,
        grid=(N // ROW_BLK,),
        in_specs=[
            pl.BlockSpec((ROW_BLK, D_IN), lambda i: (i, 0)),
            pl.BlockSpec((ROW_BLK, 1), lambda i: (i, 0)),
            pl.BlockSpec((3, D_T), lambda i: (0, 0)),
            pl.BlockSpec((D_IN, D_CAT), lambda i: (0, 0)),
            pl.BlockSpec((D_T, D_CAT), lambda i: (0, 0)),
            pl.BlockSpec((1, D_CAT), lambda i: (0, 0)),
        ],
        out_specs=[
            pl.BlockSpec((ROW_BLK, R * D_H), lambda i: (i, 0)),
            pl.BlockSpec((ROW_BLK, D_H), lambda i: (i, 0)),
        ],
        out_shape=[
            jax.ShapeDtypeStruct((N, R * D_H), jnp.float32),
            jax.ShapeDtypeStruct((N, D_H), jnp.float32),
        ],
    )(x, nt2, node_emb, wx, wt, bias_full)

    hflat = hall.reshape(N * R, D_H)

    mesh = plsc.VectorSubcoreMesh(
        core_axis_name="c", subcore_axis_name="s",
        num_cores=NC, num_subcores=NS)

    sc_params = pltpu.CompilerParams(needs_layout_passes=False)
    part_k = pl.kernel(
        _part_body,
        out_type=[
            jax.ShapeDtypeStruct((NC * NW * CAP,), jnp.int32),
            jax.ShapeDtypeStruct((NC * NW * CAP,), jnp.int32),
            jax.ShapeDtypeStruct((NW * 8, D_H), jnp.int32),
            jax.ShapeDtypeStruct((NW, CROWS, D_H), jnp.float32),
        ],
        mesh=mesh,
        compiler_params=sc_params,
        scratch_types=[
            pltpu.VMEM((CLOAD,), jnp.int32),
            pltpu.VMEM((CLOAD,), jnp.int32),
            pltpu.VMEM((CAP,), jnp.int32),
            pltpu.VMEM((CAP,), jnp.int32),
            pltpu.VMEM((CAP,), jnp.int32),
            pltpu.VMEM((CAP,), jnp.int32),
            pltpu.VMEM((D_H,), jnp.int32),
            pltpu.VMEM((CROWS, D_H), jnp.float32),
        ],
    )
    rpart, tpart, ecnt, cnt = part_k(rowidx, dtidx)

    scale = pl.pallas_call(
        _scale_body,
        grid=(CROWS // 64,),
        in_specs=[pl.BlockSpec((NW, 64, D_H), lambda i: (0, i, 0))],
        out_specs=pl.BlockSpec((64, D_H), lambda i: (i, 0)),
        out_shape=jax.ShapeDtypeStruct((CROWS, D_H), jnp.float32),
    )(cnt)

    main_k = pl.kernel(
        _main_body,
        out_type=jax.ShapeDtypeStruct((N, D_H), jnp.float32),
        mesh=mesh,
        compiler_params=sc_params,
        scratch_types=[
            pltpu.VMEM((CROWS, D_H), jnp.float32),
            pltpu.VMEM((2, B, D_H), jnp.float32),
            pltpu.VMEM((IDXC * B,), jnp.int32),
            pltpu.VMEM((IDXC * B,), jnp.int32),
            pltpu.VMEM((B,), jnp.float32),
            pltpu.VMEM((2, B), jnp.int32),
            pltpu.VMEM((2, D_H), jnp.int32),
            pltpu.VMEM_SHARED((ACC_ROWS, D_H), jnp.float32),
            pltpu.SemaphoreType.DMA,
            pltpu.SemaphoreType.DMA,
            pltpu.SemaphoreType.DMA,
            pltpu.SemaphoreType.DMA,
        ],
    )
    part = main_k(hflat, rpart, tpart, ecnt, scale)

    out = pl.pallas_call(
        _ep_body,
        grid=(N // ROW_BLK,),
        in_specs=[
            pl.BlockSpec((ROW_BLK, D_H), lambda i: (i, 0)),
            pl.BlockSpec((ROW_BLK, D_H), lambda i: (i, 0)),
        ],
        out_specs=pl.BlockSpec((ROW_BLK, D_H), lambda i: (i, 0)),
        out_shape=jax.ShapeDtypeStruct((N, D_H), jnp.float32),
    )(part, root)
    return (out,)
